# baseline (speedup 1.0000x reference)
import functools

import jax
import jax.numpy as jnp
from jax.experimental import pallas as pl
from jax.experimental.pallas import tpu as pltpu

_ROWS = 400


def _fc1_body(x_ref, w1_ref, b1_ref, a_ref):
    a_ref[...] = (
        jnp.dot(x_ref[...], w1_ref[...], preferred_element_type=jnp.float32)
        + b1_ref[...]
    )


def _pass1_body(adj_ref, a_ref, w2_ref, b2_ref, h1_ref, bm_ref):
    h1 = jnp.maximum(
        jnp.dot(adj_ref[...], a_ref[...], preferred_element_type=jnp.float32), 0.0
    )
    h1_ref[...] = h1
    bm_ref[...] = (
        jnp.dot(h1, w2_ref[...], preferred_element_type=jnp.float32) + b2_ref[...]
    )


def _pass2_body(adj_ref, bm_ref, h1_ref, watt_ref, batt_ref, wcls_ref, bcls_ref,
                out_ref, m_ref, d_ref, g_ref, *, nblk):
    i = pl.program_id(0)

    @pl.when(i == 0)
    def _init():
        m_ref[0, 0] = -jnp.inf
        d_ref[0, 0] = 0.0
        g_ref[...] = jnp.zeros_like(g_ref)

    h2 = jnp.maximum(
        jnp.dot(adj_ref[...], bm_ref[...], preferred_element_type=jnp.float32), 0.0
    )
    h = h1_ref[...] + h2
    s = (
        jnp.dot(h, watt_ref[...], preferred_element_type=jnp.float32)
        + batt_ref[0, 0]
    )

    m_old = m_ref[0, 0]
    m_new = jnp.maximum(m_old, jnp.max(s))
    scale = jnp.exp(m_old - m_new)
    e = jnp.exp(s - m_new)
    d_ref[0, 0] = d_ref[0, 0] * scale + jnp.sum(e)
    g_ref[...] = g_ref[...] * scale + jnp.sum(e * h, axis=0, keepdims=True)
    m_ref[0, 0] = m_new

    @pl.when(i == nblk - 1)
    def _fini():
        g = g_ref[...] / d_ref[0, 0]
        out_ref[...] = (
            jnp.dot(g, wcls_ref[...], preferred_element_type=jnp.float32)
            + bcls_ref[...]
        )


def kernel(x, adj, W1, b1, W2, b2, Watt, batt, Wcls, bcls):
    N, DIN = x.shape
    H = W1.shape[1]
    C = Wcls.shape[1]
    R = _ROWS
    nblk = N // R
    f32 = jnp.float32

    a = pl.pallas_call(
        _fc1_body,
        out_shape=jax.ShapeDtypeStruct((N, H), f32),
    )(x, W1, b1.reshape(1, H))

    h1, bm = pl.pallas_call(
        _pass1_body,
        grid=(nblk,),
        in_specs=[
            pl.BlockSpec((R, N), lambda i: (i, 0)),
            pl.BlockSpec((N, H), lambda i: (0, 0)),
            pl.BlockSpec((H, H), lambda i: (0, 0)),
            pl.BlockSpec((1, H), lambda i: (0, 0)),
        ],
        out_specs=[
            pl.BlockSpec((R, H), lambda i: (i, 0)),
            pl.BlockSpec((R, H), lambda i: (i, 0)),
        ],
        out_shape=[
            jax.ShapeDtypeStruct((N, H), f32),
            jax.ShapeDtypeStruct((N, H), f32),
        ],
    )(adj, a, W2, b2.reshape(1, H))

    out = pl.pallas_call(
        functools.partial(_pass2_body, nblk=nblk),
        grid=(nblk,),
        in_specs=[
            pl.BlockSpec((R, N), lambda i: (i, 0)),
            pl.BlockSpec((N, H), lambda i: (0, 0)),
            pl.BlockSpec((R, H), lambda i: (i, 0)),
            pl.BlockSpec((H, 1), lambda i: (0, 0)),
            pl.BlockSpec((1, 1), lambda i: (0, 0)),
            pl.BlockSpec((H, C), lambda i: (0, 0)),
            pl.BlockSpec((1, C), lambda i: (0, 0)),
        ],
        out_specs=pl.BlockSpec((1, C), lambda i: (0, 0)),
        out_shape=jax.ShapeDtypeStruct((1, C), f32),
        scratch_shapes=[
            pltpu.SMEM((1, 1), f32),
            pltpu.SMEM((1, 1), f32),
            pltpu.VMEM((1, H), f32),
        ],
    )(adj, bm, h1, Watt, batt.reshape(1, 1), Wcls, bcls.reshape(1, C))

    return out.reshape(C)


# speedup vs baseline: 1.0697x; 1.0697x over previous
import functools

import jax
import jax.numpy as jnp
from jax import lax
from jax.experimental import pallas as pl
from jax.experimental.pallas import tpu as pltpu

_ROWS = 400


def _body(x_ref, adj_ref, w1_ref, b1_ref, w2_ref, b2_ref, watt_ref, batt_ref,
          wcls_ref, bcls_ref, out_ref,
          a_ref, h1_ref, bm_ref, m_ref, d_ref, g_ref, *, nblk, r):
    t = pl.program_id(0)

    @pl.when(t == 0)
    def _init():
        a_ref[...] = (
            jnp.dot(x_ref[...], w1_ref[...], preferred_element_type=jnp.float32)
            + b1_ref[...]
        )
        m_ref[0, 0] = -jnp.inf
        d_ref[0, 0] = 0.0
        g_ref[...] = jnp.zeros_like(g_ref)

    @pl.when(t < nblk)
    def _pass1():
        h1 = jnp.maximum(
            jnp.dot(adj_ref[...], a_ref[...], preferred_element_type=jnp.float32),
            0.0,
        )
        h1_ref[pl.ds(t * r, r), :] = h1
        bm_ref[pl.ds(t * r, r), :] = (
            jnp.dot(h1, w2_ref[...], preferred_element_type=jnp.float32)
            + b2_ref[...]
        )

    @pl.when(t >= nblk)
    def _pass2():
        i = t - nblk
        h2 = jnp.maximum(
            jnp.dot(adj_ref[...], bm_ref[...], preferred_element_type=jnp.float32),
            0.0,
        )
        h = h1_ref[pl.ds(i * r, r), :] + h2
        s = (
            jnp.dot(h, watt_ref[...], preferred_element_type=jnp.float32)
            + batt_ref[0, 0]
        )

        m_old = m_ref[0, 0]
        m_new = jnp.maximum(m_old, jnp.max(s))
        scale = jnp.exp(m_old - m_new)
        e = jnp.exp(s - m_new)
        d_ref[0, 0] = d_ref[0, 0] * scale + jnp.sum(e)
        g_ref[...] = g_ref[...] * scale + jnp.sum(e * h, axis=0, keepdims=True)
        m_ref[0, 0] = m_new

        @pl.when(t == 2 * nblk - 1)
        def _fini():
            g = g_ref[...] / d_ref[0, 0]
            out_ref[...] = (
                jnp.dot(g, wcls_ref[...], preferred_element_type=jnp.float32)
                + bcls_ref[...]
            )


def kernel(x, adj, W1, b1, W2, b2, Watt, batt, Wcls, bcls):
    N, DIN = x.shape
    H = W1.shape[1]
    C = Wcls.shape[1]
    R = _ROWS
    nblk = N // R
    f32 = jnp.float32

    const = lambda t: (0, 0)
    out = pl.pallas_call(
        functools.partial(_body, nblk=nblk, r=R),
        grid=(2 * nblk,),
        in_specs=[
            pl.BlockSpec((N, DIN), const),
            pl.BlockSpec((R, N), lambda t: (lax.rem(t, nblk), 0)),
            pl.BlockSpec((DIN, H), const),
            pl.BlockSpec((1, H), const),
            pl.BlockSpec((H, H), const),
            pl.BlockSpec((1, H), const),
            pl.BlockSpec((H, 1), const),
            pl.BlockSpec((1, 1), const),
            pl.BlockSpec((H, C), const),
            pl.BlockSpec((1, C), const),
        ],
        out_specs=pl.BlockSpec((1, C), const),
        out_shape=jax.ShapeDtypeStruct((1, C), f32),
        scratch_shapes=[
            pltpu.VMEM((N, H), f32),
            pltpu.VMEM((N, H), f32),
            pltpu.VMEM((N, H), f32),
            pltpu.SMEM((1, 1), f32),
            pltpu.SMEM((1, 1), f32),
            pltpu.VMEM((1, H), f32),
        ],
    )(x, adj, W1, b1.reshape(1, H), W2, b2.reshape(1, H), Watt,
      batt.reshape(1, 1), Wcls, bcls.reshape(1, C))

    return out.reshape(C)
